# baseline (speedup 1.0000x reference)
import functools

import jax
import jax.numpy as jnp
from jax import lax
from jax.experimental import pallas as pl
from jax.experimental.pallas import tpu as pltpu
from jax.experimental.pallas import tpu_sc as plsc

_NC, _NS, _L = 2, 16, 16
_NW = _NC * _NS
_C = 32


def kernel(x, pe_table):
    B, S, D = x.shape
    rows_per_w = S // _NW
    nchunks = rows_per_w // _C
    nvec = D // _L
    mesh = plsc.VectorSubcoreMesh(core_axis_name="c", subcore_axis_name="s")

    @functools.partial(
        pl.kernel,
        out_type=jax.ShapeDtypeStruct((B, S, D), x.dtype),
        mesh=mesh,
        scratch_types=[
            pltpu.VMEM((_C, D), jnp.float32),
            pltpu.VMEM((_C, D), jnp.float32),
        ],
    )
    def sc_k(x_hbm, pe_hbm, out_hbm, pebuf, xbuf):
        cid = lax.axis_index("c")
        sid = lax.axis_index("s")
        wid = sid * _NC + cid
        seq0 = wid * rows_per_w

        def chunk_body(j, carry):
            row = seq0 + j * _C
            pltpu.sync_copy(pe_hbm.at[pl.ds(row, _C)], pebuf)

            def batch_body(b, carry2):
                pltpu.sync_copy(x_hbm.at[b, pl.ds(row, _C)], xbuf)

                def row_body(r, carry3):
                    for c in range(nvec):
                        pv = pebuf[r, pl.ds(c * _L, _L)]
                        plsc.addupdate(xbuf.at[r, pl.ds(c * _L, _L)], pv)
                    return carry3

                lax.fori_loop(0, _C, row_body, 0)
                pltpu.sync_copy(xbuf, out_hbm.at[b, pl.ds(row, _C)])
                return carry2

            lax.fori_loop(0, B, batch_body, 0)
            return carry

        lax.fori_loop(0, nchunks, chunk_body, 0)

    return sc_k(x, pe_table)


# speedup vs baseline: 1.1820x; 1.1820x over previous
import functools

import jax
import jax.numpy as jnp
from jax import lax
from jax.experimental import pallas as pl
from jax.experimental.pallas import tpu as pltpu
from jax.experimental.pallas import tpu_sc as plsc

_NC, _NS, _L = 2, 16, 16
_NW = _NC * _NS
_C = 16


def kernel(x, pe_table):
    B, S, D = x.shape
    rows_per_w = S // _NW
    nchunks = rows_per_w // _C
    nvec = D // _L
    nsteps_win = 2 * B
    mesh = plsc.VectorSubcoreMesh(core_axis_name="c", subcore_axis_name="s")

    @functools.partial(
        pl.kernel,
        out_type=jax.ShapeDtypeStruct((B, S, D), x.dtype),
        mesh=mesh,
        scratch_types=[
            pltpu.VMEM((_C, D), jnp.float32),
            pltpu.VMEM((_C, D), jnp.float32),
            pltpu.VMEM((_C, D), jnp.float32),
            pltpu.VMEM((_C, D), jnp.float32),
            pltpu.SemaphoreType.DMA,
            pltpu.SemaphoreType.DMA,
            pltpu.SemaphoreType.DMA,
            pltpu.SemaphoreType.DMA,
            pltpu.SemaphoreType.DMA,
            pltpu.SemaphoreType.DMA,
        ],
    )
    def sc_k(x_hbm, pe_hbm, out_hbm, xb0, xb1, pb0, pb1,
             sx0, sx1, spe0, spe1, so0, so1):
        cid = lax.axis_index("c")
        sid = lax.axis_index("s")
        wid = sid * _NC + cid
        seq0 = wid * rows_per_w

        xb = (xb0, xb1)
        pb = (pb0, pb1)
        sx = (sx0, sx1)
        spe = (spe0, spe1)
        so = (so0, so1)

        def add_chunk(pbuf, xbuf):
            def row_body(r, carry):
                for c in range(nvec):
                    pv = pbuf[r, pl.ds(c * _L, _L)]
                    plsc.addupdate(xbuf.at[r, pl.ds(c * _L, _L)], pv)
                return carry

            lax.fori_loop(0, _C, row_body, 0)

        def pe_rows(j):
            return pe_hbm.at[pl.ds(seq0 + j * _C, _C)]

        def x_rows(j, b):
            return x_hbm.at[b, pl.ds(seq0 + j * _C, _C)]

        def out_rows(j, b):
            return out_hbm.at[b, pl.ds(seq0 + j * _C, _C)]

        pltpu.async_copy(pe_rows(0), pb[0], spe[0])
        pltpu.async_copy(x_rows(0, 0), xb[0], sx[0])

        def window(jj, carry):
            sx_d = [None, None]
            so_d = [None, None]
            pe_d = [None]
            for t in range(nsteps_win):
                pj, b = divmod(t, B)
                j = jj + pj
                q = t % 2
                qn = 1 - q
                if t == 0:
                    pltpu.async_copy(pe_rows(jj + 1), pb[1], spe[1])
                    pltpu.make_async_copy(pe_rows(j), pb[0], spe[0]).wait()
                elif t == B:
                    @pl.when(jj + 2 < nchunks)
                    def _():
                        pe_d[0] = pltpu.async_copy(
                            pe_rows(jj + 2), pb[0], spe[0]
                        )
                    pltpu.make_async_copy(pe_rows(j), pb[1], spe[1]).wait()
                if t == 0:
                    @pl.when(jj > 0)
                    def _():
                        pltpu.make_async_copy(
                            xb[qn], out_rows(j, 0), so[qn]
                        ).wait()
                    sx_d[qn] = pltpu.async_copy(x_rows(j, 1), xb[qn], sx[qn])
                elif t < nsteps_win - 1:
                    so_d[qn].wait()
                    jn, bn = divmod(t + 1, B)
                    sx_d[qn] = pltpu.async_copy(
                        x_rows(jj + jn, bn), xb[qn], sx[qn]
                    )
                else:
                    so_d[qn].wait()

                    @pl.when(jj + 2 < nchunks)
                    def _():
                        sx_d[qn] = pltpu.async_copy(
                            x_rows(jj + 2, 0), xb[qn], sx[qn]
                        )
                if sx_d[q] is not None:
                    sx_d[q].wait()
                else:
                    pltpu.make_async_copy(x_rows(j, b), xb[q], sx[q]).wait()
                add_chunk(pb[pj], xb[q])
                so_d[q] = pltpu.async_copy(xb[q], out_rows(j, b), so[q])
            return carry

        lax.fori_loop(0, nchunks // 2, lambda w, c: window(2 * w, c), 0)
        pltpu.make_async_copy(
            xb[1], out_rows(nchunks - 1, B - 1), so[1]
        ).wait()

    return sc_k(x, pe_table)


# speedup vs baseline: 2.4206x; 2.0478x over previous
import functools

import jax
import jax.numpy as jnp
from jax import lax
from jax.experimental import pallas as pl
from jax.experimental.pallas import tpu as pltpu
from jax.experimental.pallas import tpu_sc as plsc

_NC, _NS, _L = 2, 16, 16
_NW = _NC * _NS
_C = 16
_G = 8
_NXB = 4


def kernel(x, pe_table):
    B, S, D = x.shape
    rows_per_w = S // _NW
    nchunks = rows_per_w // _C
    nvec = D // _L
    nsteps_win = 2 * B
    mesh = plsc.VectorSubcoreMesh(core_axis_name="c", subcore_axis_name="s")

    @functools.partial(
        pl.kernel,
        out_type=jax.ShapeDtypeStruct((B, S, D), x.dtype),
        mesh=mesh,
        scratch_types=(
            [pltpu.VMEM((_C, D), jnp.float32) for _ in range(_NXB + 2)]
            + [pltpu.SemaphoreType.DMA for _ in range(2 * _NXB + 2)]
        ),
    )
    def sc_k(x_hbm, pe_hbm, out_hbm, *bufs_and_sems):
        xb = bufs_and_sems[:_NXB]
        pb = bufs_and_sems[_NXB:_NXB + 2]
        sx = bufs_and_sems[_NXB + 2:2 * _NXB + 2]
        spe = bufs_and_sems[2 * _NXB + 2:2 * _NXB + 4]
        so = bufs_and_sems[2 * _NXB + 4:3 * _NXB + 4]

        cid = lax.axis_index("c")
        sid = lax.axis_index("s")
        wid = sid * _NC + cid
        seq0 = wid * rows_per_w

        def add_chunk(pbuf, xbuf):
            def row_body(r, carry):
                for g in range(nvec // _G):
                    pvs = [
                        pbuf[r, pl.ds((g * _G + k) * _L, _L)]
                        for k in range(_G)
                    ]
                    for k in range(_G):
                        plsc.addupdate(
                            xbuf.at[r, pl.ds((g * _G + k) * _L, _L)], pvs[k]
                        )
                return carry

            lax.fori_loop(0, _C, row_body, 0)

        def pe_rows(j):
            return pe_hbm.at[pl.ds(seq0 + j * _C, _C)]

        def x_rows(j, b):
            return x_hbm.at[b, pl.ds(seq0 + j * _C, _C)]

        def out_rows(j, b):
            return out_hbm.at[b, pl.ds(seq0 + j * _C, _C)]

        pltpu.async_copy(pe_rows(0), pb[0], spe[0])
        pltpu.async_copy(x_rows(0, 0), xb[0], sx[0])

        def window(jj, carry):
            sx_d = [None] * _NXB
            so_d = [None] * _NXB
            for t in range(nsteps_win):
                pj, b = divmod(t, B)
                j = jj + pj
                q = t % _NXB
                qn = (t + 1) % _NXB
                if t == 0:
                    pltpu.async_copy(pe_rows(jj + 1), pb[1], spe[1])
                    pltpu.make_async_copy(pe_rows(j), pb[0], spe[0]).wait()
                elif t == B:
                    @pl.when(jj + 2 < nchunks)
                    def _():
                        pltpu.async_copy(pe_rows(jj + 2), pb[0], spe[0])
                    pltpu.make_async_copy(pe_rows(j), pb[1], spe[1]).wait()
                jn, bn = divmod(t + 1, B)
                if t >= 3:
                    so_d[qn].wait()
                else:
                    @pl.when(jj > 0)
                    def _():
                        pltpu.make_async_copy(
                            xb[qn], out_rows(j, bn), so[qn]
                        ).wait()
                if t < nsteps_win - 1:
                    sx_d[qn] = pltpu.async_copy(
                        x_rows(jj + jn, bn), xb[qn], sx[qn]
                    )
                else:
                    @pl.when(jj + 2 < nchunks)
                    def _():
                        sx_d[qn] = pltpu.async_copy(
                            x_rows(jj + 2, 0), xb[qn], sx[qn]
                        )
                if sx_d[q] is not None:
                    sx_d[q].wait()
                else:
                    pltpu.make_async_copy(x_rows(j, b), xb[q], sx[q]).wait()
                add_chunk(pb[pj], xb[q])
                so_d[q] = pltpu.async_copy(xb[q], out_rows(j, b), so[q])
            return carry

        lax.fori_loop(0, nchunks // 2, lambda w, c: window(2 * w, c), 0)
        for t in range(nsteps_win - 3, nsteps_win):
            pltpu.make_async_copy(
                xb[t % _NXB], out_rows(nchunks - 1, t - B), so[t % _NXB]
            ).wait()

    return sc_k(x, pe_table)
